# Initial kernel scaffold; baseline (speedup 1.0000x reference)
import functools
import math

import jax
import jax.numpy as jnp
from jax.experimental import pallas as pl

NC = 80
TOP_K = 13
BETA = 6.0
EPS = 1e-09
IOU_EPS = 1e-07
BIG = jnp.int32(2**30)


def _assigner_kernel(ps_ref, pbt_ref, anct_ref, lab_ref, gtb_ref, mgt_ref,
                     tb_ref, ts_ref, fg_ref):
    M = lab_ref.shape[1]
    A = ps_ref.shape[1]

    ps = ps_ref[0]
    pbt = pbt_ref[0]
    gtb = gtb_ref[0]
    lab = lab_ref[0]
    mgt = mgt_ref[0]

    iota_c = jax.lax.broadcasted_iota(jnp.int32, (M, NC), 1)
    onehot_lab = (lab == iota_c).astype(jnp.float32)
    scores_full = jax.lax.dot_general(
        onehot_lab, ps, (((1,), (1,)), ((), ())),
        preferred_element_type=jnp.float32)

    ax = anct_ref[0:1, :]
    ay = anct_ref[1:2, :]
    gx1 = gtb[:, 0:1]
    gy1 = gtb[:, 1:2]
    gx2 = gtb[:, 2:3]
    gy2 = gtb[:, 3:4]
    mask_in_gts = ((ax - gx1 > EPS) & (ay - gy1 > EPS)
                   & (gx2 - ax > EPS) & (gy2 - ay > EPS))
    gt_mask = mask_in_gts & (mgt != 0.0)

    px1 = pbt[0:1, :]
    py1 = pbt[1:2, :]
    px2 = pbt[2:3, :]
    py2 = pbt[3:4, :]
    w1 = gx2 - gx1
    h1 = gy2 - gy1 + IOU_EPS
    w2 = px2 - px1
    h2 = py2 - py1 + IOU_EPS
    inter = (jnp.maximum(jnp.minimum(gx2, px2) - jnp.maximum(gx1, px1), 0.0)
             * jnp.maximum(jnp.minimum(gy2, py2) - jnp.maximum(gy1, py1), 0.0))
    union = w1 * h1 + w2 * h2 - inter + IOU_EPS
    iou = inter / union
    cw = jnp.maximum(gx2, px2) - jnp.minimum(gx1, px1)
    ch = jnp.maximum(gy2, py2) - jnp.minimum(gy1, py1)
    c2 = cw * cw + ch * ch + IOU_EPS
    t1 = (px1 + px2 - gx1 - gx2) ** 2
    t2 = (py1 + py2 - gy1 - gy2) ** 2
    rho2 = (t1 + t2) * 0.25
    datan = jnp.arctan(w2 / h2) - jnp.arctan(w1 / h1)
    v = (4.0 / math.pi**2) * (datan * datan)
    alpha = v / (v - iou + (1.0 + EPS))
    ciou = iou - (rho2 / c2 + v * alpha)

    zeros = jnp.zeros((M, A), jnp.float32)
    bbox_scores = jnp.where(gt_mask, scores_full, zeros)
    overlaps = jnp.where(gt_mask, jnp.maximum(ciou, 0.0), zeros)
    o2 = overlaps * overlaps
    align = bbox_scores * (o2 * o2 * o2)

    iota_a = jax.lax.broadcasted_iota(jnp.int32, (M, A), 1)
    work = align
    mask_top_k = jnp.zeros((M, A), jnp.bool_)
    for _ in range(TOP_K):
        rowmax = jnp.max(work, axis=1, keepdims=True)
        cand = work == rowmax
        amin = jnp.min(jnp.where(cand, iota_a, BIG), axis=1, keepdims=True)
        sel = iota_a == amin
        mask_top_k = mask_top_k | sel
        work = jnp.where(sel, -1.0, work)

    mask_pos = (mask_top_k.astype(jnp.float32)
                * mask_in_gts.astype(jnp.float32) * mgt)
    fg = jnp.sum(mask_pos, axis=0, keepdims=True)

    iota_m = jax.lax.broadcasted_iota(jnp.int32, (M, A), 0)
    colmax = jnp.max(overlaps, axis=0, keepdims=True)
    omin = jnp.min(jnp.where(overlaps == colmax, iota_m, BIG),
                   axis=0, keepdims=True)
    is_max = (iota_m == omin).astype(jnp.float32)
    mask_pos = jnp.where(fg > 1.0, is_max, mask_pos)
    fg = jnp.sum(mask_pos, axis=0, keepdims=True)

    pmax = jnp.max(mask_pos, axis=0, keepdims=True)
    tg = jnp.min(jnp.where(mask_pos == pmax, iota_m, BIG),
                 axis=0, keepdims=True)
    onehot_t = (iota_m == tg).astype(jnp.float32)

    tb_ref[0] = jax.lax.dot_general(
        onehot_t, gtb, (((0,), (0,)), ((), ())),
        preferred_element_type=jnp.float32)

    am = align * mask_pos
    pos_align = jnp.max(am, axis=1, keepdims=True)
    pos_ovl = jnp.max(overlaps * mask_pos, axis=1, keepdims=True)
    norm = jnp.max(am * pos_ovl / (pos_align + EPS),
                   axis=0, keepdims=True)
    coeff = jnp.where(fg > 0.0, norm, 0.0)

    ts_ref[0] = jax.lax.dot_general(
        onehot_t * coeff, onehot_lab, (((0,), (0,)), ((), ())),
        preferred_element_type=jnp.float32)
    fg_ref[0] = fg


@jax.jit
def kernel(pd_scores, pd_bboxes, anc_points, gt_labels, gt_bboxes, mask_gt):
    B, A, _ = pd_scores.shape
    M = gt_bboxes.shape[1]
    pbt = jnp.transpose(pd_bboxes, (0, 2, 1))
    anct = anc_points.T
    lab = gt_labels.astype(jnp.int32)
    mgt = mask_gt.astype(jnp.float32)

    tb, ts, fg = pl.pallas_call(
        _assigner_kernel,
        grid=(B,),
        in_specs=[
            pl.BlockSpec((1, A, NC), lambda b: (b, 0, 0)),
            pl.BlockSpec((1, 4, A), lambda b: (b, 0, 0)),
            pl.BlockSpec((2, A), lambda b: (0, 0)),
            pl.BlockSpec((1, M, 1), lambda b: (b, 0, 0)),
            pl.BlockSpec((1, M, 4), lambda b: (b, 0, 0)),
            pl.BlockSpec((1, M, 1), lambda b: (b, 0, 0)),
        ],
        out_specs=[
            pl.BlockSpec((1, A, 4), lambda b: (b, 0, 0)),
            pl.BlockSpec((1, A, NC), lambda b: (b, 0, 0)),
            pl.BlockSpec((1, 1, A), lambda b: (b, 0, 0)),
        ],
        out_shape=[
            jax.ShapeDtypeStruct((B, A, 4), jnp.float32),
            jax.ShapeDtypeStruct((B, A, NC), jnp.float32),
            jax.ShapeDtypeStruct((B, 1, A), jnp.float32),
        ],
    )(pd_scores, pbt, anct, lab, gt_bboxes, mgt)

    return tb, ts, (fg[:, 0, :] != 0.0)


# speedup vs baseline: 19.0188x; 19.0188x over previous
import functools
import math

import jax
import jax.numpy as jnp
from jax.experimental import pallas as pl

NC = 80
TOP_K = 13
BETA = 6.0
EPS = 1e-09
IOU_EPS = 1e-07
BIG = 2**30


_ATAN_COEFFS = (0.0028662257, -0.0161657367, 0.0429096138, -0.0752896400,
                0.1065626393, -0.1420889944, 0.1999355085, -0.3333314528)


def _atan(x):
    t = jnp.abs(x)
    inv = t > 1.0
    z = jnp.where(inv, 1.0 / t, t)
    z2 = z * z
    p = jnp.full_like(z, _ATAN_COEFFS[0])
    for c in _ATAN_COEFFS[1:]:
        p = p * z2 + c
    p = p * z2 + 1.0
    r = z * p
    r = jnp.where(inv, (math.pi / 2) - r, r)
    return jnp.where(x < 0, -r, r)


def _assigner_kernel(ps_ref, pbt_ref, anct_ref, lab_ref, gtb_ref, mgt_ref,
                     tb_ref, ts_ref, fg_ref):
    M = lab_ref.shape[1]
    A = ps_ref.shape[1]

    ps = ps_ref[0]
    pbt = pbt_ref[0]
    gtb = gtb_ref[0]
    lab = lab_ref[0]
    mgt = mgt_ref[0]

    iota_c = jax.lax.broadcasted_iota(jnp.int32, (M, NC), 1)
    onehot_lab = (lab == iota_c).astype(jnp.float32)
    scores_full = jax.lax.dot_general(
        onehot_lab, ps, (((1,), (1,)), ((), ())),
        preferred_element_type=jnp.float32)

    ax = anct_ref[0:1, :]
    ay = anct_ref[1:2, :]
    gx1 = gtb[:, 0:1]
    gy1 = gtb[:, 1:2]
    gx2 = gtb[:, 2:3]
    gy2 = gtb[:, 3:4]
    mask_in_gts = ((ax - gx1 > EPS) & (ay - gy1 > EPS)
                   & (gx2 - ax > EPS) & (gy2 - ay > EPS))
    gt_mask = mask_in_gts & (mgt != 0.0)

    px1 = pbt[0:1, :]
    py1 = pbt[1:2, :]
    px2 = pbt[2:3, :]
    py2 = pbt[3:4, :]
    w1 = gx2 - gx1
    h1 = gy2 - gy1 + IOU_EPS
    w2 = px2 - px1
    h2 = py2 - py1 + IOU_EPS
    inter = (jnp.maximum(jnp.minimum(gx2, px2) - jnp.maximum(gx1, px1), 0.0)
             * jnp.maximum(jnp.minimum(gy2, py2) - jnp.maximum(gy1, py1), 0.0))
    union = w1 * h1 + w2 * h2 - inter + IOU_EPS
    iou = inter / union
    cw = jnp.maximum(gx2, px2) - jnp.minimum(gx1, px1)
    ch = jnp.maximum(gy2, py2) - jnp.minimum(gy1, py1)
    c2 = cw * cw + ch * ch + IOU_EPS
    t1 = (px1 + px2 - gx1 - gx2) ** 2
    t2 = (py1 + py2 - gy1 - gy2) ** 2
    rho2 = (t1 + t2) * 0.25
    datan = _atan(w2 / h2) - _atan(w1 / h1)
    v = (4.0 / math.pi**2) * (datan * datan)
    alpha = v / (v - iou + (1.0 + EPS))
    ciou = iou - (rho2 / c2 + v * alpha)

    zeros = jnp.zeros((M, A), jnp.float32)
    bbox_scores = jnp.where(gt_mask, scores_full, zeros)
    overlaps = jnp.where(gt_mask, jnp.maximum(ciou, 0.0), zeros)
    o2 = overlaps * overlaps
    align = bbox_scores * (o2 * o2 * o2)

    iota_a = jax.lax.broadcasted_iota(jnp.int32, (M, A), 1)
    work = align
    mask_top_k = jnp.zeros((M, A), jnp.bool_)
    for _ in range(TOP_K):
        rowmax = jnp.max(work, axis=1, keepdims=True)
        cand = work == rowmax
        amin = jnp.min(jnp.where(cand, iota_a, BIG), axis=1, keepdims=True)
        sel = iota_a == amin
        mask_top_k = mask_top_k | sel
        work = jnp.where(sel, -1.0, work)

    mask_pos = (mask_top_k.astype(jnp.float32)
                * mask_in_gts.astype(jnp.float32) * mgt)
    fg = jnp.sum(mask_pos, axis=0, keepdims=True)

    iota_m = jax.lax.broadcasted_iota(jnp.int32, (M, A), 0)
    colmax = jnp.max(overlaps, axis=0, keepdims=True)
    omin = jnp.min(jnp.where(overlaps == colmax, iota_m, BIG),
                   axis=0, keepdims=True)
    is_max = (iota_m == omin).astype(jnp.float32)
    mask_pos = jnp.where(fg > 1.0, is_max, mask_pos)
    fg = jnp.sum(mask_pos, axis=0, keepdims=True)

    pmax = jnp.max(mask_pos, axis=0, keepdims=True)
    tg = jnp.min(jnp.where(mask_pos == pmax, iota_m, BIG),
                 axis=0, keepdims=True)
    onehot_t = (iota_m == tg).astype(jnp.float32)

    tb_ref[0] = jax.lax.dot_general(
        onehot_t, gtb, (((0,), (0,)), ((), ())),
        preferred_element_type=jnp.float32)

    am = align * mask_pos
    pos_align = jnp.max(am, axis=1, keepdims=True)
    pos_ovl = jnp.max(overlaps * mask_pos, axis=1, keepdims=True)
    norm = jnp.max(am * pos_ovl / (pos_align + EPS),
                   axis=0, keepdims=True)
    coeff = jnp.where(fg > 0.0, norm, 0.0)

    ts_ref[0] = jax.lax.dot_general(
        onehot_t * coeff, onehot_lab, (((0,), (0,)), ((), ())),
        preferred_element_type=jnp.float32)
    fg_ref[0] = fg


@jax.jit
def kernel(pd_scores, pd_bboxes, anc_points, gt_labels, gt_bboxes, mask_gt):
    B, A, _ = pd_scores.shape
    M = gt_bboxes.shape[1]
    pbt = jnp.transpose(pd_bboxes, (0, 2, 1))
    anct = anc_points.T
    lab = gt_labels.astype(jnp.int32)
    mgt = mask_gt.astype(jnp.float32)

    tb, ts, fg = pl.pallas_call(
        _assigner_kernel,
        grid=(B,),
        in_specs=[
            pl.BlockSpec((1, A, NC), lambda b: (b, 0, 0)),
            pl.BlockSpec((1, 4, A), lambda b: (b, 0, 0)),
            pl.BlockSpec((2, A), lambda b: (0, 0)),
            pl.BlockSpec((1, M, 1), lambda b: (b, 0, 0)),
            pl.BlockSpec((1, M, 4), lambda b: (b, 0, 0)),
            pl.BlockSpec((1, M, 1), lambda b: (b, 0, 0)),
        ],
        out_specs=[
            pl.BlockSpec((1, A, 4), lambda b: (b, 0, 0)),
            pl.BlockSpec((1, A, NC), lambda b: (b, 0, 0)),
            pl.BlockSpec((1, 1, A), lambda b: (b, 0, 0)),
        ],
        out_shape=[
            jax.ShapeDtypeStruct((B, A, 4), jnp.float32),
            jax.ShapeDtypeStruct((B, A, NC), jnp.float32),
            jax.ShapeDtypeStruct((B, 1, A), jnp.float32),
        ],
    )(pd_scores, pbt, anct, lab, gt_bboxes, mgt)

    return tb, ts, (fg[:, 0, :] != 0.0)


# speedup vs baseline: 20.0867x; 1.0562x over previous
import functools
import math

import jax
import jax.numpy as jnp
from jax.experimental import pallas as pl

NC = 80
TOP_K = 13
BETA = 6.0
EPS = 1e-09
IOU_EPS = 1e-07
BIG = 2**30


_ATAN_COEFFS = (0.0028662257, -0.0161657367, 0.0429096138, -0.0752896400,
                0.1065626393, -0.1420889944, 0.1999355085, -0.3333314528)


def _atan(x):
    t = jnp.abs(x)
    inv = t > 1.0
    z = jnp.where(inv, 1.0 / t, t)
    z2 = z * z
    p = jnp.full_like(z, _ATAN_COEFFS[0])
    for c in _ATAN_COEFFS[1:]:
        p = p * z2 + c
    p = p * z2 + 1.0
    r = z * p
    r = jnp.where(inv, (math.pi / 2) - r, r)
    return jnp.where(x < 0, -r, r)


def _assigner_kernel(ps_ref, pbt_ref, anct_ref, lab_ref, gtb_ref, mgt_ref,
                     tb_ref, ts_ref, fg_ref):
    M = lab_ref.shape[1]
    A = ps_ref.shape[1]

    ps = ps_ref[0]
    pbt = pbt_ref[0]
    gtb = gtb_ref[0]
    lab = lab_ref[0]
    mgt = mgt_ref[0]

    iota_c = jax.lax.broadcasted_iota(jnp.int32, (M, NC), 1)
    onehot_lab = (lab == iota_c).astype(jnp.float32)
    scores_full = jax.lax.dot_general(
        onehot_lab, ps, (((1,), (1,)), ((), ())),
        preferred_element_type=jnp.float32)

    ax = anct_ref[0:1, :]
    ay = anct_ref[1:2, :]
    gx1 = gtb[:, 0:1]
    gy1 = gtb[:, 1:2]
    gx2 = gtb[:, 2:3]
    gy2 = gtb[:, 3:4]
    mask_in_gts = ((ax - gx1 > EPS) & (ay - gy1 > EPS)
                   & (gx2 - ax > EPS) & (gy2 - ay > EPS))
    gt_mask = mask_in_gts & (mgt != 0.0)

    px1 = pbt[0:1, :]
    py1 = pbt[1:2, :]
    px2 = pbt[2:3, :]
    py2 = pbt[3:4, :]
    w1 = gx2 - gx1
    h1 = gy2 - gy1 + IOU_EPS
    w2 = px2 - px1
    h2 = py2 - py1 + IOU_EPS
    inter = (jnp.maximum(jnp.minimum(gx2, px2) - jnp.maximum(gx1, px1), 0.0)
             * jnp.maximum(jnp.minimum(gy2, py2) - jnp.maximum(gy1, py1), 0.0))
    union = w1 * h1 + w2 * h2 - inter + IOU_EPS
    iou = inter / union
    cw = jnp.maximum(gx2, px2) - jnp.minimum(gx1, px1)
    ch = jnp.maximum(gy2, py2) - jnp.minimum(gy1, py1)
    c2 = cw * cw + ch * ch + IOU_EPS
    t1 = (px1 + px2 - gx1 - gx2) ** 2
    t2 = (py1 + py2 - gy1 - gy2) ** 2
    rho2 = (t1 + t2) * 0.25
    datan = _atan(w2 / h2) - _atan(w1 / h1)
    v = (4.0 / math.pi**2) * (datan * datan)
    d = v - iou + (1.0 + EPS)
    ciou = iou - (rho2 * d + v * v * c2) / (c2 * d)

    zeros = jnp.zeros((M, A), jnp.float32)
    bbox_scores = jnp.where(gt_mask, scores_full, zeros)
    overlaps = jnp.where(gt_mask, jnp.maximum(ciou, 0.0), zeros)
    o2 = overlaps * overlaps
    align = bbox_scores * (o2 * o2 * o2)

    iota_a = jax.lax.broadcasted_iota(jnp.int32, (M, A), 1)
    work = align
    for _ in range(TOP_K):
        rowmax = jnp.max(work, axis=1, keepdims=True)
        amin = jnp.min(jnp.where(work == rowmax, iota_a, BIG),
                       axis=1, keepdims=True)
        work = jnp.where(iota_a == amin, -1.0, work)

    mask_pos = (jnp.where(work < 0.0, 1.0, 0.0)
                * mask_in_gts.astype(jnp.float32) * mgt)
    fg = jnp.sum(mask_pos, axis=0, keepdims=True)

    iota_m = jax.lax.broadcasted_iota(jnp.int32, (M, A), 0)
    colmax = jnp.max(overlaps, axis=0, keepdims=True)
    omin = jnp.min(jnp.where(overlaps == colmax, iota_m, BIG),
                   axis=0, keepdims=True)
    is_max = (iota_m == omin).astype(jnp.float32)
    mask_pos = jnp.where(fg > 1.0, is_max, mask_pos)
    fg = jnp.sum(mask_pos, axis=0, keepdims=True)

    pmax = jnp.max(mask_pos, axis=0, keepdims=True)
    tg = jnp.min(jnp.where(mask_pos == pmax, iota_m, BIG),
                 axis=0, keepdims=True)
    onehot_t = (iota_m == tg).astype(jnp.float32)

    tb_ref[0] = jax.lax.dot_general(
        onehot_t, gtb, (((0,), (0,)), ((), ())),
        preferred_element_type=jnp.float32)

    am = align * mask_pos
    pos_align = jnp.max(am, axis=1, keepdims=True)
    pos_ovl = jnp.max(overlaps * mask_pos, axis=1, keepdims=True)
    norm = jnp.max(am * pos_ovl / (pos_align + EPS),
                   axis=0, keepdims=True)
    coeff = jnp.where(fg > 0.0, norm, 0.0)

    ts_ref[0] = jax.lax.dot_general(
        onehot_t * coeff, onehot_lab, (((0,), (0,)), ((), ())),
        preferred_element_type=jnp.float32)
    fg_ref[0] = fg


@jax.jit
def kernel(pd_scores, pd_bboxes, anc_points, gt_labels, gt_bboxes, mask_gt):
    B, A, _ = pd_scores.shape
    M = gt_bboxes.shape[1]
    pbt = jnp.transpose(pd_bboxes, (0, 2, 1))
    anct = anc_points.T
    lab = gt_labels.astype(jnp.int32)
    mgt = mask_gt.astype(jnp.float32)

    tb, ts, fg = pl.pallas_call(
        _assigner_kernel,
        grid=(B,),
        in_specs=[
            pl.BlockSpec((1, A, NC), lambda b: (b, 0, 0)),
            pl.BlockSpec((1, 4, A), lambda b: (b, 0, 0)),
            pl.BlockSpec((2, A), lambda b: (0, 0)),
            pl.BlockSpec((1, M, 1), lambda b: (b, 0, 0)),
            pl.BlockSpec((1, M, 4), lambda b: (b, 0, 0)),
            pl.BlockSpec((1, M, 1), lambda b: (b, 0, 0)),
        ],
        out_specs=[
            pl.BlockSpec((1, A, 4), lambda b: (b, 0, 0)),
            pl.BlockSpec((1, A, NC), lambda b: (b, 0, 0)),
            pl.BlockSpec((1, 1, A), lambda b: (b, 0, 0)),
        ],
        out_shape=[
            jax.ShapeDtypeStruct((B, A, 4), jnp.float32),
            jax.ShapeDtypeStruct((B, A, NC), jnp.float32),
            jax.ShapeDtypeStruct((B, 1, A), jnp.float32),
        ],
    )(pd_scores, pbt, anct, lab, gt_bboxes, mgt)

    return tb, ts, (fg[:, 0, :] != 0.0)


# speedup vs baseline: 20.4124x; 1.0162x over previous
import functools
import math

import jax
import jax.numpy as jnp
from jax.experimental import pallas as pl

NC = 80
TOP_K = 13
BETA = 6.0
EPS = 1e-09
IOU_EPS = 1e-07
BIG = 2**30


_ATAN_COEFFS = (0.0028662257, -0.0161657367, 0.0429096138, -0.0752896400,
                0.1065626393, -0.1420889944, 0.1999355085, -0.3333314528)


def _atan(x):
    t = jnp.abs(x)
    inv = t > 1.0
    z = jnp.where(inv, 1.0 / t, t)
    z2 = z * z
    p = jnp.full_like(z, _ATAN_COEFFS[0])
    for c in _ATAN_COEFFS[1:]:
        p = p * z2 + c
    p = p * z2 + 1.0
    r = z * p
    r = jnp.where(inv, (math.pi / 2) - r, r)
    return jnp.where(x < 0, -r, r)


def _assigner_kernel(ps_ref, pbt_ref, anct_ref, lab_ref, gtb_ref, mgt_ref,
                     tb_ref, ts_ref, fg_ref):
    M = lab_ref.shape[1]
    A = ps_ref.shape[1]

    ps = ps_ref[0]
    pbt = pbt_ref[0]
    gtb = gtb_ref[0]
    lab = lab_ref[0]
    mgt = mgt_ref[0]

    iota_c = jax.lax.broadcasted_iota(jnp.int32, (M, NC), 1)
    onehot_lab = (lab == iota_c).astype(jnp.float32)
    scores_full = jax.lax.dot_general(
        onehot_lab, ps, (((1,), (1,)), ((), ())),
        preferred_element_type=jnp.float32)

    ax = anct_ref[0:1, :]
    ay = anct_ref[1:2, :]
    gx1 = gtb[:, 0:1]
    gy1 = gtb[:, 1:2]
    gx2 = gtb[:, 2:3]
    gy2 = gtb[:, 3:4]
    mask_in_gts = ((ax > gx1 + EPS) & (ay > gy1 + EPS)
                   & (gx2 - EPS > ax) & (gy2 - EPS > ay))
    gt_mask = mask_in_gts & (mgt != 0.0)

    px1 = pbt[0:1, :]
    py1 = pbt[1:2, :]
    px2 = pbt[2:3, :]
    py2 = pbt[3:4, :]
    w1 = gx2 - gx1
    h1 = gy2 - gy1 + IOU_EPS
    w2 = px2 - px1
    h2 = py2 - py1 + IOU_EPS
    inter = (jnp.maximum(jnp.minimum(gx2, px2) - jnp.maximum(gx1, px1), 0.0)
             * jnp.maximum(jnp.minimum(gy2, py2) - jnp.maximum(gy1, py1), 0.0))
    area1 = w1 * h1
    area2 = w2 * h2
    union = (area1 + area2) - inter + IOU_EPS
    iou = inter / union
    cw = jnp.maximum(gx2, px2) - jnp.minimum(gx1, px1)
    ch = jnp.maximum(gy2, py2) - jnp.minimum(gy1, py1)
    c2 = cw * cw + ch * ch + IOU_EPS
    spx = px1 + px2
    spy = py1 + py2
    sgx = gx1 + gx2
    sgy = gy1 + gy2
    dx = spx - sgx
    dy = spy - sgy
    rho2 = (dx * dx + dy * dy) * 0.25
    datan = _atan(w2 / h2) - _atan(w1 / h1)
    v = (4.0 / math.pi**2) * (datan * datan)
    d = v - iou + (1.0 + EPS)
    ciou = iou - (rho2 * d + v * v * c2) / (c2 * d)

    zeros = jnp.zeros((M, A), jnp.float32)
    bbox_scores = jnp.where(gt_mask, scores_full, zeros)
    overlaps = jnp.where(gt_mask, jnp.maximum(ciou, 0.0), zeros)
    o2 = overlaps * overlaps
    align = bbox_scores * (o2 * o2 * o2)

    iota_a = jax.lax.broadcasted_iota(jnp.int32, (M, A), 1)
    work = align
    for _ in range(TOP_K):
        rowmax = jnp.max(work, axis=1, keepdims=True)
        amin = jnp.min(jnp.where(work == rowmax, iota_a, BIG),
                       axis=1, keepdims=True)
        work = jnp.where(iota_a == amin, -1.0, work)

    mask_pos = (jnp.where(work < 0.0, 1.0, 0.0)
                * mask_in_gts.astype(jnp.float32) * mgt)
    fg = jnp.sum(mask_pos, axis=0, keepdims=True)

    iota_m = jax.lax.broadcasted_iota(jnp.int32, (M, A), 0)
    colmax = jnp.max(overlaps, axis=0, keepdims=True)
    omin = jnp.min(jnp.where(overlaps == colmax, iota_m, BIG),
                   axis=0, keepdims=True)
    is_max = (iota_m == omin).astype(jnp.float32)
    mask_pos = jnp.where(fg > 1.0, is_max, mask_pos)
    fg = jnp.sum(mask_pos, axis=0, keepdims=True)

    pmax = jnp.max(mask_pos, axis=0, keepdims=True)
    tg = jnp.min(jnp.where(mask_pos == pmax, iota_m, BIG),
                 axis=0, keepdims=True)
    onehot_t = (iota_m == tg).astype(jnp.float32)

    tb_ref[0] = jax.lax.dot_general(
        onehot_t, gtb, (((0,), (0,)), ((), ())),
        preferred_element_type=jnp.float32)

    am = align * mask_pos
    pos_align = jnp.max(am, axis=1, keepdims=True)
    pos_ovl = jnp.max(overlaps * mask_pos, axis=1, keepdims=True)
    norm = jnp.max(am * pos_ovl / (pos_align + EPS),
                   axis=0, keepdims=True)
    coeff = jnp.where(fg > 0.0, norm, 0.0)

    ts_ref[0] = jax.lax.dot_general(
        onehot_t * coeff, onehot_lab, (((0,), (0,)), ((), ())),
        preferred_element_type=jnp.float32)
    fg_ref[0] = fg


@jax.jit
def kernel(pd_scores, pd_bboxes, anc_points, gt_labels, gt_bboxes, mask_gt):
    B, A, _ = pd_scores.shape
    M = gt_bboxes.shape[1]
    pbt = jnp.transpose(pd_bboxes, (0, 2, 1))
    anct = anc_points.T
    lab = gt_labels.astype(jnp.int32)
    mgt = mask_gt.astype(jnp.float32)

    tb, ts, fg = pl.pallas_call(
        _assigner_kernel,
        grid=(B,),
        in_specs=[
            pl.BlockSpec((1, A, NC), lambda b: (b, 0, 0)),
            pl.BlockSpec((1, 4, A), lambda b: (b, 0, 0)),
            pl.BlockSpec((2, A), lambda b: (0, 0)),
            pl.BlockSpec((1, M, 1), lambda b: (b, 0, 0)),
            pl.BlockSpec((1, M, 4), lambda b: (b, 0, 0)),
            pl.BlockSpec((1, M, 1), lambda b: (b, 0, 0)),
        ],
        out_specs=[
            pl.BlockSpec((1, A, 4), lambda b: (b, 0, 0)),
            pl.BlockSpec((1, A, NC), lambda b: (b, 0, 0)),
            pl.BlockSpec((1, 1, A), lambda b: (b, 0, 0)),
        ],
        out_shape=[
            jax.ShapeDtypeStruct((B, A, 4), jnp.float32),
            jax.ShapeDtypeStruct((B, A, NC), jnp.float32),
            jax.ShapeDtypeStruct((B, 1, A), jnp.float32),
        ],
    )(pd_scores, pbt, anct, lab, gt_bboxes, mgt)

    return tb, ts, (fg[:, 0, :] != 0.0)
